# Initial kernel scaffold; baseline (speedup 1.0000x reference)
import jax
import jax.numpy as jnp
from jax.experimental import pallas as pl


def kernel(x, edge_index, W1, b1, W2, b2):
    raise NotImplementedError("write your pallas kernel here")


# speedup vs baseline: 2.6635x; 2.6635x over previous
import functools

import jax
import jax.numpy as jnp
from jax import lax
from jax.experimental import pallas as pl
from jax.experimental.pallas import tpu as pltpu
from jax.experimental.pallas import tpu_sc as plsc

N_NODES = 10000
D_IN = 128
D_H = 128
N_CLS = 64

NC = 2
NS = 16
NW = NC * NS


def _mlp_body(x_ref, w1_ref, b1_ref, w2_ref, b2_ref, z_ref):
    h = jnp.dot(x_ref[...], w1_ref[...], preferred_element_type=jnp.float32)
    h = jnp.maximum(h + b1_ref[...], 0.0)
    z_ref[...] = jnp.dot(h, w2_ref[...], preferred_element_type=jnp.float32) + b2_ref[...]


def _node_mlp(x, W1, b1, W2, b2):
    n = x.shape[0]
    blk = 2000
    grid = (n // blk,)
    return pl.pallas_call(
        _mlp_body,
        grid=grid,
        in_specs=[
            pl.BlockSpec((blk, D_IN), lambda i: (i, 0)),
            pl.BlockSpec((D_IN, D_H), lambda i: (0, 0)),
            pl.BlockSpec((1, D_H), lambda i: (0, 0)),
            pl.BlockSpec((D_H, N_CLS), lambda i: (0, 0)),
            pl.BlockSpec((1, N_CLS), lambda i: (0, 0)),
        ],
        out_specs=pl.BlockSpec((blk, N_CLS), lambda i: (i, 0)),
        out_shape=jax.ShapeDtypeStruct((n, N_CLS), jnp.float32),
    )(x, W1, b1.reshape(1, D_H), W2, b2.reshape(1, N_CLS))


def _gather_rows(table, idx, chunk):
    b = idx.shape[0]
    d = table.shape[1]
    per_w = b // NW
    n_chunks = per_w // chunk
    assert per_w * NW == b and n_chunks * chunk == per_w

    mesh = plsc.VectorSubcoreMesh(core_axis_name="c", subcore_axis_name="s")

    @functools.partial(
        pl.kernel,
        mesh=mesh,
        compiler_params=pltpu.CompilerParams(use_tc_tiling_on_sc=False),
        out_type=jax.ShapeDtypeStruct((b, d), jnp.float32),
        scratch_types=[
            pltpu.VMEM((chunk,), jnp.int32),
            pltpu.VMEM((chunk, d), jnp.float32),
            pltpu.SemaphoreType.DMA,
        ],
    )
    def k(table_hbm, idx_hbm, out_hbm, idx_v, rows_v, sem):
        wid = lax.axis_index("s") * NC + lax.axis_index("c")
        base = wid * per_w

        def body(c, _):
            off = base + c * chunk
            pltpu.sync_copy(idx_hbm.at[pl.ds(off, chunk)], idx_v)
            pltpu.async_copy(table_hbm.at[idx_v], rows_v, sem).wait()
            pltpu.sync_copy(rows_v, out_hbm.at[pl.ds(off, chunk)])
            return _

        lax.fori_loop(0, n_chunks, body, 0)

    return k(table, idx)


def kernel(x, edge_index, W1, b1, W2, b2):
    col = edge_index[1]
    z = _node_mlp(x, W1, b1, W2, b2)
    colh = col[:N_NODES]
    pad = (-N_NODES) % (NW * 80)
    colh_p = jnp.concatenate([colh, jnp.zeros((pad,), jnp.int32)])
    z2 = _gather_rows(z, colh_p, chunk=80)
    return _gather_rows(z2, col, chunk=80)


# speedup vs baseline: 4.2019x; 1.5776x over previous
import functools

import jax
import jax.numpy as jnp
from jax import lax
from jax.experimental import pallas as pl
from jax.experimental.pallas import tpu as pltpu
from jax.experimental.pallas import tpu_sc as plsc

N_NODES = 10000
D_IN = 128
D_H = 128
N_CLS = 64

NC = 2
NS = 16
NW = NC * NS

CH = 128
K = 6


def _mlp_body(x_ref, w1_ref, b1_ref, w2_ref, b2_ref, z_ref):
    h = jnp.dot(x_ref[...], w1_ref[...], preferred_element_type=jnp.float32)
    h = jnp.maximum(h + b1_ref[...], 0.0)
    z_ref[...] = jnp.dot(h, w2_ref[...], preferred_element_type=jnp.float32) + b2_ref[...]


def _node_mlp(x, W1, b1, W2, b2):
    n = x.shape[0]
    blk = 2000
    grid = (n // blk,)
    return pl.pallas_call(
        _mlp_body,
        grid=grid,
        in_specs=[
            pl.BlockSpec((blk, D_IN), lambda i: (i, 0)),
            pl.BlockSpec((D_IN, D_H), lambda i: (0, 0)),
            pl.BlockSpec((1, D_H), lambda i: (0, 0)),
            pl.BlockSpec((D_H, N_CLS), lambda i: (0, 0)),
            pl.BlockSpec((1, N_CLS), lambda i: (0, 0)),
        ],
        out_specs=pl.BlockSpec((blk, N_CLS), lambda i: (i, 0)),
        out_shape=jax.ShapeDtypeStruct((n, N_CLS), jnp.float32),
    )(x, W1, b1.reshape(1, D_H), W2, b2.reshape(1, N_CLS))


def _gcn_gather(z, col):
    E = col.shape[0]
    d = z.shape[1]
    per_w = E // NW
    n_full = per_w // CH
    tail = per_w - n_full * CH
    n_groups = n_full // K
    assert per_w * NW == E and n_groups * K == n_full
    assert tail % 8 == 0 and tail in (0, 16)

    mesh = plsc.VectorSubcoreMesh(core_axis_name="c", subcore_axis_name="s")

    @functools.partial(
        pl.kernel,
        mesh=mesh,
        compiler_params=pltpu.CompilerParams(
            use_tc_tiling_on_sc=False, needs_layout_passes=False),
        out_type=jax.ShapeDtypeStruct((E, d), jnp.float32),
        scratch_types=[
            pltpu.VMEM_SHARED((N_NODES, d), jnp.float32),
            pltpu.VMEM((N_NODES,), jnp.int32),
            pltpu.VMEM((per_w,), jnp.int32),
            pltpu.VMEM((K, CH), jnp.int32),
            pltpu.VMEM((K, CH, d), jnp.float32),
            pltpu.VMEM((16,), jnp.int32),
            pltpu.VMEM((16, d), jnp.float32),
            *([pltpu.SemaphoreType.DMA] * (2 * K + 1)),
        ],
    )
    def k(z_hbm, col_hbm, out_hbm, zsh, colh_v, colw_v, ibuf, rbuf, tb_i, tb_r, *sems):
        gsems, wsems, tsem = sems[:K], sems[K:2 * K], sems[2 * K]
        wid = lax.axis_index("s") * NC + lax.axis_index("c")
        base = wid * per_w

        @pl.when(lax.axis_index("s") == 0)
        def _():
            pltpu.sync_copy(z_hbm, zsh)
        pltpu.sync_copy(col_hbm.at[pl.ds(0, N_NODES)], colh_v)
        pltpu.sync_copy(col_hbm.at[pl.ds(base, per_w)], colw_v)
        plsc.subcore_barrier()

        def compute_ibuf(slot, coff):
            for kk in range(CH // 16):
                cv = colw_v[pl.ds(coff + kk * 16, 16)]
                ibuf[slot, pl.ds(kk * 16, 16)] = plsc.load_gather(colh_v, [cv])

        def group(g, carry):
            handles = []
            for b in range(K):
                coff = (g * K + b) * CH

                @pl.when(g > 0)
                def _():
                    pltpu.make_async_copy(
                        rbuf.at[b], out_hbm.at[pl.ds(0, CH)], wsems[b]).wait()

                compute_ibuf(b, coff)
                handles.append(
                    pltpu.async_copy(zsh.at[ibuf.at[b]], rbuf.at[b], gsems[b]))
            for b in range(K):
                handles[b].wait()
                pltpu.async_copy(
                    rbuf.at[b],
                    out_hbm.at[pl.ds(base + (g * K + b) * CH, CH)],
                    wsems[b])
            return carry

        lax.fori_loop(0, n_groups, group, 0)
        for b in range(K):
            pltpu.make_async_copy(
                rbuf.at[b], out_hbm.at[pl.ds(0, CH)], wsems[b]).wait()

        if tail:
            cv = colw_v[pl.ds(n_full * CH, 16)]
            tb_i[...] = plsc.load_gather(colh_v, [cv])
            pltpu.async_copy(zsh.at[tb_i], tb_r, tsem).wait()
            pltpu.sync_copy(tb_r, out_hbm.at[pl.ds(base + n_full * CH, 16)])

    return k(z, col)


def kernel(x, edge_index, W1, b1, W2, b2):
    col = edge_index[1]
    z = _node_mlp(x, W1, b1, W2, b2)
    return _gcn_gather(z, col)
